# baseline (speedup 1.0000x reference)
import functools

import jax
import jax.numpy as jnp
from jax import lax
from jax.experimental import pallas as pl
from jax.experimental.pallas import tpu as pltpu
from jax.experimental.pallas import tpu_sc as plsc

NC = 2
NS = 16
NW = NC * NS
CB = 128


@functools.cache
def _make_gather(V, D, BATCH, SEQ):
    assert BATCH % (NW * CB) == 0 and D % 8 == 0
    BW = BATCH // NW
    NJ = BW // CB
    n_chunks = SEQ * NJ
    assert n_chunks % 2 == 0
    mesh = plsc.VectorSubcoreMesh(core_axis_name="c", subcore_axis_name="s")

    @functools.partial(
        pl.kernel,
        out_type=jax.ShapeDtypeStruct((SEQ, D // 8, BATCH // CB, 8, CB),
                                      jnp.float32),
        mesh=mesh,
        scratch_types=[
            pltpu.VMEM((SEQ, BW), jnp.int32),
            pltpu.VMEM((2, CB, D), jnp.float32),
            pltpu.VMEM((2, D // 8, 8, CB), jnp.float32),
            pltpu.SemaphoreType.DMA,
            pltpu.SemaphoreType.DMA,
            pltpu.SemaphoreType.DMA,
            pltpu.SemaphoreType.DMA,
        ],
        compiler_params=pltpu.CompilerParams(
            use_tc_tiling_on_sc=False, needs_layout_passes=False),
    )
    def gather_kernel(table_hbm, xt_hbm, out_hbm, idx_v, rows_v, tbuf,
                      sem_g0, sem_g1, sem_s0, sem_s1):
        wid = lax.axis_index("s") * NC + lax.axis_index("c")
        wb0 = wid * BW
        wbt0 = wid * NJ
        sem_g = (sem_g0, sem_g1)
        sem_s = (sem_s0, sem_s1)

        pltpu.sync_copy(xt_hbm.at[:, pl.ds(wb0, BW)], idx_v)

        iota = lax.iota(jnp.int32, 16)
        bidx = [iota + 16 * g for g in range(CB // 16)]

        def split(c):
            s = c // NJ
            j = c - s * NJ
            return s, j

        def gather_desc(c, buf):
            s, j = split(c)
            return pltpu.make_async_copy(
                table_hbm.at[idx_v.at[s, pl.ds(j * CB, CB)]],
                rows_v.at[buf],
                sem_g[buf],
            )

        def fire(c, buf):
            s, j = split(c)
            pltpu.async_copy(
                table_hbm.at[idx_v.at[s, pl.ds(j * CB, CB)]],
                rows_v.at[buf],
                sem_g[buf],
            )

        def store_desc(c, buf):
            s, j = split(c)
            return pltpu.make_async_copy(
                tbuf.at[buf],
                out_hbm.at[s, :, wbt0 + j],
                sem_s[buf],
            )

        def transpose(buf):
            rows = rows_v.at[buf]
            tb = tbuf.at[buf]
            for d in range(D):
                dvec = jnp.full((16,), d, dtype=jnp.int32)
                for g in range(CB // 16):
                    v = plsc.load_gather(rows, [bidx[g], dvec])
                    tb[d // 8, d % 8, pl.ds(16 * g, 16)] = v

        fire(0, 0)

        @pl.loop(0, n_chunks, step=2)
        def _(c0):
            for u in range(2):
                c = c0 + u

                @pl.when(c + 1 < n_chunks)
                def _():
                    fire(c + 1, 1 - u)

                gather_desc(c, u).wait()

                @pl.when(c >= 2)
                def _():
                    store_desc(c - 2, u).wait()

                transpose(u)

                s, j = split(c)
                pltpu.async_copy(
                    tbuf.at[u], out_hbm.at[s, :, wbt0 + j], sem_s[u])

        store_desc(n_chunks - 2, 0).wait()
        store_desc(n_chunks - 1, 1).wait()

    return gather_kernel


def kernel(X, emb):
    batch, seq = X.shape
    D = emb.shape[1]
    Xt = X.T.astype(jnp.int32)
    out5 = _make_gather(emb.shape[0], D, batch, seq)(emb, Xt)
    return out5.transpose(2, 4, 0, 1, 3).reshape(batch, seq, D)


# speedup vs baseline: 1.7992x; 1.7992x over previous
import functools

import jax
import jax.numpy as jnp
from jax import lax
from jax.experimental import pallas as pl
from jax.experimental.pallas import tpu as pltpu
from jax.experimental.pallas import tpu_sc as plsc

NC = 2
NS = 16
NW = NC * NS
CB = 128


@functools.cache
def _make_gather(V, D, BATCH, SEQ):
    assert BATCH % (NW * CB) == 0 and D % 8 == 0
    BW = BATCH // NW
    NJ = BW // CB
    n_chunks = SEQ * NJ
    assert n_chunks % 2 == 0
    mesh = plsc.VectorSubcoreMesh(core_axis_name="c", subcore_axis_name="s")

    @functools.partial(
        pl.kernel,
        out_type=jax.ShapeDtypeStruct((SEQ, D // 8, BATCH // CB, 8, CB),
                                      jnp.float32),
        mesh=mesh,
        scratch_types=[
            pltpu.VMEM((SEQ, BW), jnp.int32),
            pltpu.VMEM((2, CB, D), jnp.float32),
            pltpu.VMEM((2, D // 8, 8, CB + 1), jnp.float32),
            pltpu.SemaphoreType.DMA,
            pltpu.SemaphoreType.DMA,
            pltpu.SemaphoreType.DMA,
            pltpu.SemaphoreType.DMA,
        ],
        compiler_params=pltpu.CompilerParams(
            use_tc_tiling_on_sc=False, needs_layout_passes=False),
    )
    def gather_kernel(table_hbm, xt_hbm, out_hbm, idx_v, rows_v, tbuf,
                      sem_g0, sem_g1, sem_s0, sem_s1):
        wid = lax.axis_index("s") * NC + lax.axis_index("c")
        wb0 = wid * BW
        wbt0 = wid * NJ
        sem_g = (sem_g0, sem_g1)
        sem_s = (sem_s0, sem_s1)

        pltpu.sync_copy(xt_hbm.at[:, pl.ds(wb0, BW)], idx_v)

        iota = lax.iota(jnp.int32, 16)
        bidx = [iota + 16 * g for g in range(CB // 16)]

        def split(c):
            s = c // NJ
            j = c - s * NJ
            return s, j

        def gather_desc(c, buf):
            s, j = split(c)
            return pltpu.make_async_copy(
                table_hbm.at[idx_v.at[s, pl.ds(j * CB, CB)]],
                rows_v.at[buf],
                sem_g[buf],
            )

        def fire(c, buf):
            s, j = split(c)
            pltpu.async_copy(
                table_hbm.at[idx_v.at[s, pl.ds(j * CB, CB)]],
                rows_v.at[buf],
                sem_g[buf],
            )

        def store_desc(c, buf):
            s, j = split(c)
            return pltpu.make_async_copy(
                tbuf.at[buf].at[:, :, pl.ds(0, CB)],
                out_hbm.at[s, :, wbt0 + j],
                sem_s[buf],
            )

        dhi = [(16 * k + iota) // 8 for k in range(D // 16)]
        dlo = iota % 8

        def transpose(buf):
            rows = rows_v.at[buf]
            tb = tbuf.at[buf]
            for b in range(CB):
                bvec = jnp.full((16,), b, dtype=jnp.int32)
                for k in range(D // 16):
                    v = rows[b, pl.ds(16 * k, 16)]
                    plsc.store_scatter(tb, [dhi[k], dlo, bvec], v)

        fire(0, 0)

        @pl.loop(0, n_chunks, step=2)
        def _(c0):
            for u in range(2):
                c = c0 + u

                @pl.when(c + 1 < n_chunks)
                def _():
                    fire(c + 1, 1 - u)

                gather_desc(c, u).wait()

                @pl.when(c >= 2)
                def _():
                    store_desc(c - 2, u).wait()

                transpose(u)

                s, j = split(c)
                pltpu.async_copy(
                    tbuf.at[u].at[:, :, pl.ds(0, CB)],
                    out_hbm.at[s, :, wbt0 + j], sem_s[u])

        store_desc(n_chunks - 2, 0).wait()
        store_desc(n_chunks - 1, 1).wait()

    return gather_kernel


def kernel(X, emb):
    batch, seq = X.shape
    D = emb.shape[1]
    Xt = X.T.astype(jnp.int32)
    out5 = _make_gather(emb.shape[0], D, batch, seq)(emb, Xt)
    return out5.transpose(2, 4, 0, 1, 3).reshape(batch, seq, D)


# speedup vs baseline: 2.5343x; 1.4086x over previous
import functools

import jax
import jax.numpy as jnp
from jax import lax
from jax.experimental import pallas as pl
from jax.experimental.pallas import tpu as pltpu
from jax.experimental.pallas import tpu_sc as plsc

NC = 2
NS = 16
NW = NC * NS
CB = 128


@functools.cache
def _make_gather(V, D, BATCH, SEQ):
    assert BATCH % (NW * CB) == 0 and D % 8 == 0
    BW = BATCH // NW
    NJ = BW // CB
    n_chunks = SEQ * NJ
    assert n_chunks % 2 == 0
    mesh = plsc.VectorSubcoreMesh(core_axis_name="c", subcore_axis_name="s")

    @functools.partial(
        pl.kernel,
        out_type=jax.ShapeDtypeStruct((SEQ, D // 8, BATCH // CB, 8, CB),
                                      jnp.float32),
        mesh=mesh,
        scratch_types=[
            pltpu.VMEM((SEQ, BW), jnp.int32),
            pltpu.VMEM((2, CB, D), jnp.float32),
            pltpu.VMEM((2, D // 8, 8, CB + 1), jnp.float32),
            pltpu.SemaphoreType.DMA,
            pltpu.SemaphoreType.DMA,
            pltpu.SemaphoreType.DMA,
            pltpu.SemaphoreType.DMA,
        ],
        compiler_params=pltpu.CompilerParams(
            use_tc_tiling_on_sc=False, needs_layout_passes=False),
    )
    def gather_kernel(table_hbm, xt_hbm, out_hbm, idx_v, rows_v, tbuf,
                      sem_g0, sem_g1, sem_s0, sem_s1):
        wid = lax.axis_index("s") * NC + lax.axis_index("c")
        wb0 = wid * BW
        wbt0 = wid * NJ
        sem_g = (sem_g0, sem_g1)
        sem_s = (sem_s0, sem_s1)

        pltpu.sync_copy(xt_hbm.at[:, pl.ds(wb0, BW)], idx_v)

        iota = lax.iota(jnp.int32, 16)
        bidx = [iota + 16 * g for g in range(CB // 16)]

        def split(c):
            s = c // NJ
            j = c - s * NJ
            return s, j

        def gather_desc(c, buf):
            s, j = split(c)
            return pltpu.make_async_copy(
                table_hbm.at[idx_v.at[s, pl.ds(j * CB, CB)]],
                rows_v.at[buf],
                sem_g[buf],
            )

        def fire(c, buf):
            s, j = split(c)
            pltpu.async_copy(
                table_hbm.at[idx_v.at[s, pl.ds(j * CB, CB)]],
                rows_v.at[buf],
                sem_g[buf],
            )

        def store_desc(c, buf):
            s, j = split(c)
            return pltpu.make_async_copy(
                tbuf.at[buf].at[:, :, pl.ds(0, CB)],
                out_hbm.at[s, :, wbt0 + j],
                sem_s[buf],
            )

        dhi = [(16 * k + iota) // 8 for k in range(D // 16)]
        dlo = iota % 8

        zeros16 = jnp.zeros((16,), dtype=jnp.int32)

        def transpose(buf):
            rows = rows_v.at[buf]
            tb = tbuf.at[buf]

            @plsc.parallel_loop(0, CB, unroll=8)
            def _(b):
                bvec = zeros16 + b
                for k in range(D // 16):
                    v = rows[b, pl.ds(16 * k, 16)]
                    plsc.store_scatter(tb, [dhi[k], dlo, bvec], v)

        fire(0, 0)

        @pl.loop(0, n_chunks, step=2)
        def _(c0):
            for u in range(2):
                c = c0 + u

                @pl.when(c + 1 < n_chunks)
                def _():
                    fire(c + 1, 1 - u)

                gather_desc(c, u).wait()

                @pl.when(c >= 2)
                def _():
                    store_desc(c - 2, u).wait()

                transpose(u)

                s, j = split(c)
                pltpu.async_copy(
                    tbuf.at[u].at[:, :, pl.ds(0, CB)],
                    out_hbm.at[s, :, wbt0 + j], sem_s[u])

        store_desc(n_chunks - 2, 0).wait()
        store_desc(n_chunks - 1, 1).wait()

    return gather_kernel


def kernel(X, emb):
    batch, seq = X.shape
    D = emb.shape[1]
    Xt = X.T.astype(jnp.int32)
    out5 = _make_gather(emb.shape[0], D, batch, seq)(emb, Xt)
    return out5.transpose(2, 4, 0, 1, 3).reshape(batch, seq, D)


# speedup vs baseline: 2.6499x; 1.0456x over previous
import functools

import jax
import jax.numpy as jnp
from jax import lax
from jax.experimental import pallas as pl
from jax.experimental.pallas import tpu as pltpu
from jax.experimental.pallas import tpu_sc as plsc

NC = 2
NS = 16
NW = NC * NS
CB = 128


@functools.cache
def _make_gather(V, D, BATCH, SEQ):
    assert BATCH % (NW * CB) == 0 and D % 8 == 0
    BW = BATCH // NW
    NJ = BW // CB
    n_chunks = SEQ * NJ
    assert n_chunks % 2 == 0
    mesh = plsc.VectorSubcoreMesh(core_axis_name="c", subcore_axis_name="s")

    @functools.partial(
        pl.kernel,
        out_type=jax.ShapeDtypeStruct((SEQ, D // 8, BATCH // CB, 8, CB),
                                      jnp.float32),
        mesh=mesh,
        scratch_types=[
            pltpu.VMEM((SEQ, BW), jnp.int32),
            pltpu.VMEM((4, CB, D), jnp.float32),
            pltpu.VMEM((2, D // 8, 8, CB + 1), jnp.float32),
            pltpu.SemaphoreType.DMA,
            pltpu.SemaphoreType.DMA,
            pltpu.SemaphoreType.DMA,
            pltpu.SemaphoreType.DMA,
            pltpu.SemaphoreType.DMA,
            pltpu.SemaphoreType.DMA,
        ],
        compiler_params=pltpu.CompilerParams(
            use_tc_tiling_on_sc=False, needs_layout_passes=False),
    )
    def gather_kernel(table_hbm, xt_hbm, out_hbm, idx_v, rows_v, tbuf,
                      sem_g0, sem_g1, sem_g2, sem_g3, sem_s0, sem_s1):
        wid = lax.axis_index("s") * NC + lax.axis_index("c")
        wb0 = wid * BW
        wbt0 = wid * NJ
        sem_g = (sem_g0, sem_g1, sem_g2, sem_g3)
        sem_s = (sem_s0, sem_s1)

        pltpu.sync_copy(xt_hbm.at[:, pl.ds(wb0, BW)], idx_v)

        iota = lax.iota(jnp.int32, 16)
        bidx = [iota + 16 * g for g in range(CB // 16)]

        def split(c):
            s = c // NJ
            j = c - s * NJ
            return s, j

        def gather_desc(c, buf):
            s, j = split(c)
            return pltpu.make_async_copy(
                table_hbm.at[idx_v.at[s, pl.ds(j * CB, CB)]],
                rows_v.at[buf],
                sem_g[buf],
            )

        def fire(c, buf):
            s, j = split(c)
            pltpu.async_copy(
                table_hbm.at[idx_v.at[s, pl.ds(j * CB, CB)]],
                rows_v.at[buf],
                sem_g[buf],
            )

        def store_desc(c, buf):
            s, j = split(c)
            return pltpu.make_async_copy(
                tbuf.at[buf].at[:, :, pl.ds(0, CB)],
                out_hbm.at[s, :, wbt0 + j],
                sem_s[buf],
            )

        dhi = [(16 * k + iota) // 8 for k in range(D // 16)]
        dlo = iota % 8

        zeros16 = jnp.zeros((16,), dtype=jnp.int32)

        def transpose(buf, tbi):
            rows = rows_v.at[buf]
            tb = tbuf.at[tbi]

            @plsc.parallel_loop(0, CB, unroll=8)
            def _(b):
                bvec = zeros16 + b
                for k in range(D // 16):
                    v = rows[b, pl.ds(16 * k, 16)]
                    plsc.store_scatter(tb, [dhi[k], dlo, bvec], v)

        fire(0, 0)
        fire(1, 1)

        @pl.loop(0, n_chunks, step=4)
        def _(c0):
            for r in range(4):
                c = c0 + r
                u = r
                t = r % 2

                @pl.when(c + 2 < n_chunks)
                def _():
                    fire(c + 2, (r + 2) % 4)

                gather_desc(c, u).wait()

                @pl.when(c >= 2)
                def _():
                    store_desc(c - 2, t).wait()

                transpose(u, t)

                s, j = split(c)
                pltpu.async_copy(
                    tbuf.at[t].at[:, :, pl.ds(0, CB)],
                    out_hbm.at[s, :, wbt0 + j], sem_s[t])

        store_desc(n_chunks - 2, 0).wait()
        store_desc(n_chunks - 1, 1).wait()

    return gather_kernel


def kernel(X, emb):
    batch, seq = X.shape
    D = emb.shape[1]
    Xt = X.T.astype(jnp.int32)
    out5 = _make_gather(emb.shape[0], D, batch, seq)(emb, Xt)
    return out5.transpose(2, 4, 0, 1, 3).reshape(batch, seq, D)


# speedup vs baseline: 2.6609x; 1.0042x over previous
import functools

import jax
import jax.numpy as jnp
from jax import lax
from jax.experimental import pallas as pl
from jax.experimental.pallas import tpu as pltpu
from jax.experimental.pallas import tpu_sc as plsc

NC = 2
NS = 16
NW = NC * NS
CB = 128


@functools.cache
def _make_gather(V, D, BATCH, SEQ):
    assert BATCH % (NW * CB) == 0 and D % 8 == 0
    BW = BATCH // NW
    NJ = BW // CB
    n_chunks = SEQ * NJ
    assert n_chunks % 2 == 0
    mesh = plsc.VectorSubcoreMesh(core_axis_name="c", subcore_axis_name="s")

    @functools.partial(
        pl.kernel,
        out_type=jax.ShapeDtypeStruct((SEQ, D // 8, BATCH // CB, 8, CB),
                                      jnp.float32),
        mesh=mesh,
        scratch_types=[
            pltpu.VMEM((SEQ, BW), jnp.int32),
            pltpu.VMEM((4, CB, D), jnp.float32),
            pltpu.VMEM((2, D // 8, 8, CB + 1), jnp.float32),
            pltpu.SemaphoreType.DMA,
            pltpu.SemaphoreType.DMA,
            pltpu.SemaphoreType.DMA,
            pltpu.SemaphoreType.DMA,
            pltpu.SemaphoreType.DMA,
            pltpu.SemaphoreType.DMA,
        ],
        compiler_params=pltpu.CompilerParams(
            use_tc_tiling_on_sc=False, needs_layout_passes=False),
    )
    def gather_kernel(table_hbm, xt_hbm, out_hbm, idx_v, rows_v, tbuf,
                      sem_g0, sem_g1, sem_g2, sem_g3, sem_s0, sem_s1):
        wid = lax.axis_index("s") * NC + lax.axis_index("c")
        wb0 = wid * BW
        wbt0 = wid * NJ
        sem_g = (sem_g0, sem_g1, sem_g2, sem_g3)
        sem_s = (sem_s0, sem_s1)

        pltpu.sync_copy(xt_hbm.at[:, pl.ds(wb0, BW)], idx_v)

        iota = lax.iota(jnp.int32, 16)
        bidx = [iota + 16 * g for g in range(CB // 16)]

        def split(c):
            s = c // NJ
            j = c - s * NJ
            return s, j

        def gather_desc(c, buf):
            s, j = split(c)
            return pltpu.make_async_copy(
                table_hbm.at[idx_v.at[s, pl.ds(j * CB, CB)]],
                rows_v.at[buf],
                sem_g[buf],
            )

        def fire(c, buf):
            s, j = split(c)
            pltpu.async_copy(
                table_hbm.at[idx_v.at[s, pl.ds(j * CB, CB)]],
                rows_v.at[buf],
                sem_g[buf],
            )

        def store_desc(c, buf):
            s, j = split(c)
            return pltpu.make_async_copy(
                tbuf.at[buf].at[:, :, pl.ds(0, CB)],
                out_hbm.at[s, :, wbt0 + j],
                sem_s[buf],
            )

        dhi = [(16 * k + iota) // 8 for k in range(D // 16)]
        dlo = iota % 8

        zeros16 = jnp.zeros((16,), dtype=jnp.int32)

        def transpose(buf, tbi):
            rows = rows_v.at[buf]
            tb = tbuf.at[tbi]

            @plsc.parallel_loop(0, CB, unroll=8)
            def _(b):
                bvec = zeros16 + b
                for k in range(D // 16):
                    v = rows[b, pl.ds(16 * k, 16)]
                    plsc.store_scatter(tb, [dhi[k], dlo, bvec], v)

        fire(0, 0)
        fire(1, 1)
        fire(2, 2)

        @pl.loop(0, n_chunks, step=4)
        def _(c0):
            for r in range(4):
                c = c0 + r
                u = r
                t = r % 2

                @pl.when(c + 3 < n_chunks)
                def _():
                    fire(c + 3, (r + 3) % 4)

                gather_desc(c, u).wait()

                @pl.when(c >= 2)
                def _():
                    store_desc(c - 2, t).wait()

                transpose(u, t)

                s, j = split(c)
                pltpu.async_copy(
                    tbuf.at[t].at[:, :, pl.ds(0, CB)],
                    out_hbm.at[s, :, wbt0 + j], sem_s[t])

        store_desc(n_chunks - 2, 0).wait()
        store_desc(n_chunks - 1, 1).wait()

    return gather_kernel


def kernel(X, emb):
    batch, seq = X.shape
    D = emb.shape[1]
    Xt = X.T.astype(jnp.int32)
    out5 = _make_gather(emb.shape[0], D, batch, seq)(emb, Xt)
    return out5.transpose(2, 4, 0, 1, 3).reshape(batch, seq, D)
